# initial kernel scaffold (unmeasured)
import jax
import jax.numpy as jnp
from jax import lax
from jax.experimental import pallas as pl
from jax.experimental.pallas import tpu as pltpu

N_DEV = 4


def kernel(x, w_mat, scale_x, scale_w):
    m, _ = x.shape
    _, n = w_mat.shape
    mc = m // N_DEV

    xb = x.astype(jnp.bfloat16)
    wb = w_mat.astype(jnp.bfloat16)

    def body(x_ref, w_ref, sx_ref, sw_ref, out_ref, comm_ref,
             send_sems, recv_sems):
        me = lax.axis_index("i")
        left = (me + N_DEV - 1) % N_DEV
        right = (me + 1) % N_DEV

        barrier_sem = pltpu.get_barrier_semaphore()
        for nbr in (left, right):
            pl.semaphore_signal(
                barrier_sem, inc=1,
                device_id=(nbr,), device_id_type=pl.DeviceIdType.MESH,
            )
        pl.semaphore_wait(barrier_sem, 2)

        out_ref[...] = jnp.dot(
            x_ref[...], w_ref[...], preferred_element_type=jnp.float32
        )

        for h in range(N_DEV - 1):
            slot = h % 2
            send_c = (me - h) % N_DEV
            recv_c = (me - h - 1) % N_DEV
            rdma = pltpu.make_async_remote_copy(
                src_ref=out_ref.at[pl.ds(send_c * mc, mc), :],
                dst_ref=comm_ref.at[slot],
                send_sem=send_sems.at[slot],
                recv_sem=recv_sems.at[slot],
                device_id=(right,),
                device_id_type=pl.DeviceIdType.MESH,
            )
            rdma.start()
            rdma.wait()
            out_ref[pl.ds(recv_c * mc, mc), :] = (
                out_ref[pl.ds(recv_c * mc, mc), :] + comm_ref[slot]
            )

        own_c = (me + 1) % N_DEV
        scale = sx_ref[0] * sw_ref[0]
        y = out_ref[pl.ds(own_c * mc, mc), :] * scale
        out_ref[pl.ds(own_c * mc, mc), :] = y * jax.nn.sigmoid(y)

        for s in range(N_DEV - 1):
            slot = (N_DEV - 1 + s) % 2
            send_c = (me + 1 - s) % N_DEV
            recv_c = (me - s) % N_DEV
            rdma = pltpu.make_async_remote_copy(
                src_ref=out_ref.at[pl.ds(send_c * mc, mc), :],
                dst_ref=comm_ref.at[slot],
                send_sem=send_sems.at[slot],
                recv_sem=recv_sems.at[slot],
                device_id=(right,),
                device_id_type=pl.DeviceIdType.MESH,
            )
            rdma.start()
            rdma.wait()
            out_ref[pl.ds(recv_c * mc, mc), :] = comm_ref[slot]

    return pl.pallas_call(
        body,
        out_shape=jax.ShapeDtypeStruct((m, n), jnp.float32),
        in_specs=[
            pl.BlockSpec(memory_space=pltpu.VMEM),
            pl.BlockSpec(memory_space=pltpu.VMEM),
            pl.BlockSpec(memory_space=pltpu.SMEM),
            pl.BlockSpec(memory_space=pltpu.SMEM),
        ],
        out_specs=pl.BlockSpec(memory_space=pltpu.VMEM),
        scratch_shapes=[
            pltpu.VMEM((2, mc, n), jnp.float32),
            pltpu.SemaphoreType.DMA((2,)),
            pltpu.SemaphoreType.DMA((2,)),
        ],
        compiler_params=pltpu.CompilerParams(collective_id=0),
    )(xb, wb, scale_x, scale_w)


# baseline (device time: 352576 ns/iter reference)
import jax
import jax.numpy as jnp
from jax import lax
from jax.experimental import pallas as pl
from jax.experimental.pallas import tpu as pltpu

N_DEV = 4


def kernel(x, w_mat, scale_x, scale_w):
    m, _ = x.shape
    _, n = w_mat.shape
    mc = m // N_DEV

    xb = x.astype(jnp.bfloat16)
    wb = w_mat.astype(jnp.bfloat16)

    def body(x_ref, w_ref, sx_ref, sw_ref, out_hbm, acc, comm, stage,
             send_sems, recv_sems, copy_sems):
        me = lax.axis_index("i")
        left = (me + N_DEV - 1) % N_DEV
        right = (me + 1) % N_DEV

        barrier_sem = pltpu.get_barrier_semaphore()
        for nbr in (left, right):
            pl.semaphore_signal(
                barrier_sem, inc=1,
                device_id=(nbr,), device_id_type=pl.DeviceIdType.MESH,
            )
        pl.semaphore_wait(barrier_sem, 2)

        for c in range(N_DEV):
            acc[pl.ds(c * mc, mc), :] = jnp.dot(
                x_ref[pl.ds(c * mc, mc), :], w_ref[...],
                preferred_element_type=jnp.float32,
            ).astype(jnp.bfloat16)

        for h in range(N_DEV - 1):
            slot = h % 2
            send_c = (me - h) % N_DEV
            recv_c = (me - h - 1) % N_DEV
            rdma = pltpu.make_async_remote_copy(
                src_ref=acc.at[pl.ds(send_c * mc, mc), :],
                dst_ref=comm.at[slot],
                send_sem=send_sems.at[slot],
                recv_sem=recv_sems.at[slot],
                device_id=(right,),
                device_id_type=pl.DeviceIdType.MESH,
            )
            rdma.start()
            rdma.wait()
            acc[pl.ds(recv_c * mc, mc), :] = (
                acc[pl.ds(recv_c * mc, mc), :] + comm[slot]
            )

        own_c = (me + 1) % N_DEV
        scale = sx_ref[0] * sw_ref[0]
        y = acc[pl.ds(own_c * mc, mc), :].astype(jnp.float32) * scale
        z = y * jax.nn.sigmoid(y)
        acc[pl.ds(own_c * mc, mc), :] = z.astype(jnp.bfloat16)
        stage[0] = z
        own_copy = pltpu.make_async_copy(
            stage.at[0], out_hbm.at[pl.ds(own_c * mc, mc), :],
            copy_sems.at[0],
        )
        own_copy.start()

        for s in range(N_DEV - 1):
            slot = (N_DEV - 1 + s) % 2
            cslot = (s + 1) % 2
            send_c = (me + 1 - s) % N_DEV
            recv_c = (me - s) % N_DEV
            rdma = pltpu.make_async_remote_copy(
                src_ref=acc.at[pl.ds(send_c * mc, mc), :],
                dst_ref=comm.at[slot],
                send_sem=send_sems.at[slot],
                recv_sem=recv_sems.at[slot],
                device_id=(right,),
                device_id_type=pl.DeviceIdType.MESH,
            )
            rdma.start()
            rdma.wait()
            acc[pl.ds(recv_c * mc, mc), :] = comm[slot]
            if s >= 1:
                pltpu.make_async_copy(
                    stage.at[cslot], out_hbm.at[pl.ds(recv_c * mc, mc), :],
                    copy_sems.at[cslot],
                ).wait()
            stage[cslot] = comm[slot].astype(jnp.float32)
            pltpu.make_async_copy(
                stage.at[cslot], out_hbm.at[pl.ds(recv_c * mc, mc), :],
                copy_sems.at[cslot],
            ).start()

        for cslot in (0, 1):
            pltpu.make_async_copy(
                stage.at[cslot], out_hbm.at[pl.ds(0, mc), :],
                copy_sems.at[cslot],
            ).wait()

    return pl.pallas_call(
        body,
        out_shape=jax.ShapeDtypeStruct((m, n), jnp.float32),
        in_specs=[
            pl.BlockSpec(memory_space=pltpu.VMEM),
            pl.BlockSpec(memory_space=pltpu.VMEM),
            pl.BlockSpec(memory_space=pltpu.SMEM),
            pl.BlockSpec(memory_space=pltpu.SMEM),
        ],
        out_specs=pl.BlockSpec(memory_space=pl.ANY),
        scratch_shapes=[
            pltpu.VMEM((m, n), jnp.bfloat16),
            pltpu.VMEM((2, mc, n), jnp.bfloat16),
            pltpu.VMEM((2, mc, n), jnp.float32),
            pltpu.SemaphoreType.DMA((2,)),
            pltpu.SemaphoreType.DMA((2,)),
            pltpu.SemaphoreType.DMA((2,)),
        ],
        compiler_params=pltpu.CompilerParams(
            collective_id=0,
            vmem_limit_bytes=60 * 1024 * 1024,
        ),
    )(xb, wb, scale_x, scale_w)


# device time: 204118 ns/iter; 1.7273x vs baseline; 1.7273x over previous
import jax
import jax.numpy as jnp
from jax import lax
from jax.experimental import pallas as pl
from jax.experimental.pallas import tpu as pltpu

N_DEV = 4


def kernel(x, w_mat, scale_x, scale_w):
    m, _ = x.shape
    _, n = w_mat.shape
    mc = m // N_DEV
    hf = mc // 2

    xb = x.astype(jnp.bfloat16)
    wb = w_mat.astype(jnp.bfloat16)

    def body(x_ref, w_ref, sx_ref, sw_ref, out_hbm, acc, comm_r, comm_l,
             stage_r, stage_l, send_r, recv_r, send_l, recv_l,
             copy_r, copy_l):
        me = lax.axis_index("i")
        left = (me + N_DEV - 1) % N_DEV
        right = (me + 1) % N_DEV

        def rows_r(c):
            return pl.ds(c * mc, hf)

        def rows_l(c):
            return pl.ds(c * mc + hf, hf)

        barrier_sem = pltpu.get_barrier_semaphore()
        for nbr in (left, right):
            pl.semaphore_signal(
                barrier_sem, inc=1,
                device_id=(nbr,), device_id_type=pl.DeviceIdType.MESH,
            )
        pl.semaphore_wait(barrier_sem, 2)

        def gemm_chunk(c):
            acc[pl.ds(c * mc, mc), :] = jnp.dot(
                x_ref[pl.ds(c * mc, mc), :], w_ref[...],
                preferred_element_type=jnp.float32,
            ).astype(jnp.bfloat16)

        gemm_chunk(me)

        for h in range(N_DEV - 1):
            slot = h % 2
            rdma_r = pltpu.make_async_remote_copy(
                src_ref=acc.at[rows_r((me - h) % N_DEV), :],
                dst_ref=comm_r.at[slot],
                send_sem=send_r.at[slot],
                recv_sem=recv_r.at[slot],
                device_id=(right,),
                device_id_type=pl.DeviceIdType.MESH,
            )
            rdma_l = pltpu.make_async_remote_copy(
                src_ref=acc.at[rows_l((me + h) % N_DEV), :],
                dst_ref=comm_l.at[slot],
                send_sem=send_l.at[slot],
                recv_sem=recv_l.at[slot],
                device_id=(left,),
                device_id_type=pl.DeviceIdType.MESH,
            )
            rdma_r.start()
            rdma_l.start()
            if h == 0:
                for k in range(1, N_DEV):
                    gemm_chunk((me + k) % N_DEV)
            rdma_r.wait()
            rdma_l.wait()
            rc_r = (me - h - 1) % N_DEV
            rc_l = (me + h + 1) % N_DEV
            acc[rows_r(rc_r), :] = acc[rows_r(rc_r), :] + comm_r[slot]
            acc[rows_l(rc_l), :] = acc[rows_l(rc_l), :] + comm_l[slot]

        scale = sx_ref[0] * sw_ref[0]
        own_r = (me + 1) % N_DEV
        own_l = (me - 1) % N_DEV
        for rows, stage, csem, own in (
            (rows_r(own_r), stage_r, copy_r, own_r),
            (rows_l(own_l), stage_l, copy_l, own_l),
        ):
            y = acc[rows, :].astype(jnp.float32) * scale
            z = y * jax.nn.sigmoid(y)
            acc[rows, :] = z.astype(jnp.bfloat16)
            stage[0] = z
            pltpu.make_async_copy(
                stage.at[0], out_hbm.at[rows, :], csem.at[0],
            ).start()

        for s in range(N_DEV - 1):
            slot = (N_DEV - 1 + s) % 2
            cslot = (s + 1) % 2
            rdma_r = pltpu.make_async_remote_copy(
                src_ref=acc.at[rows_r((me + 1 - s) % N_DEV), :],
                dst_ref=comm_r.at[slot],
                send_sem=send_r.at[slot],
                recv_sem=recv_r.at[slot],
                device_id=(right,),
                device_id_type=pl.DeviceIdType.MESH,
            )
            rdma_l = pltpu.make_async_remote_copy(
                src_ref=acc.at[rows_l((me - 1 + s) % N_DEV), :],
                dst_ref=comm_l.at[slot],
                send_sem=send_l.at[slot],
                recv_sem=recv_l.at[slot],
                device_id=(left,),
                device_id_type=pl.DeviceIdType.MESH,
            )
            rdma_r.start()
            rdma_l.start()
            rdma_r.wait()
            rdma_l.wait()
            rc_r = (me - s) % N_DEV
            rc_l = (me + s) % N_DEV
            acc[rows_r(rc_r), :] = comm_r[slot]
            acc[rows_l(rc_l), :] = comm_l[slot]
            for rows, stage, csem, cm in (
                (rows_r(rc_r), stage_r, copy_r, comm_r),
                (rows_l(rc_l), stage_l, copy_l, comm_l),
            ):
                cp = pltpu.make_async_copy(
                    stage.at[cslot], out_hbm.at[rows, :], csem.at[cslot],
                )
                if s >= 1:
                    cp.wait()
                stage[cslot] = cm[slot].astype(jnp.float32)
                cp.start()

        for cslot in (0, 1):
            for stage, csem in ((stage_r, copy_r), (stage_l, copy_l)):
                pltpu.make_async_copy(
                    stage.at[cslot], out_hbm.at[pl.ds(0, hf), :],
                    csem.at[cslot],
                ).wait()

    return pl.pallas_call(
        body,
        out_shape=jax.ShapeDtypeStruct((m, n), jnp.float32),
        in_specs=[
            pl.BlockSpec(memory_space=pltpu.VMEM),
            pl.BlockSpec(memory_space=pltpu.VMEM),
            pl.BlockSpec(memory_space=pltpu.SMEM),
            pl.BlockSpec(memory_space=pltpu.SMEM),
        ],
        out_specs=pl.BlockSpec(memory_space=pl.ANY),
        scratch_shapes=[
            pltpu.VMEM((m, n), jnp.bfloat16),
            pltpu.VMEM((2, hf, n), jnp.bfloat16),
            pltpu.VMEM((2, hf, n), jnp.bfloat16),
            pltpu.VMEM((2, hf, n), jnp.float32),
            pltpu.VMEM((2, hf, n), jnp.float32),
            pltpu.SemaphoreType.DMA((2,)),
            pltpu.SemaphoreType.DMA((2,)),
            pltpu.SemaphoreType.DMA((2,)),
            pltpu.SemaphoreType.DMA((2,)),
            pltpu.SemaphoreType.DMA((2,)),
            pltpu.SemaphoreType.DMA((2,)),
        ],
        compiler_params=pltpu.CompilerParams(
            collective_id=0,
            vmem_limit_bytes=62 * 1024 * 1024,
        ),
    )(xb, wb, scale_x, scale_w)


# device time: 201723 ns/iter; 1.7478x vs baseline; 1.0119x over previous
import jax
import jax.numpy as jnp
from jax import lax
from jax.experimental import pallas as pl
from jax.experimental.pallas import tpu as pltpu

N_DEV = 4


def kernel(x, w_mat, scale_x, scale_w):
    m, _ = x.shape
    _, n = w_mat.shape
    mc = m // N_DEV
    hf = mc // 2

    xb = x.astype(jnp.bfloat16)
    wb = w_mat.astype(jnp.bfloat16)

    def body(x_ref, w_ref, sx_ref, sw_ref, out_hbm, acc, comm_r, comm_l,
             stage_r, stage_l, send_r, recv_r, send_l, recv_l,
             copy_r, copy_l):
        me = lax.axis_index("i")
        left = (me + N_DEV - 1) % N_DEV
        right = (me + 1) % N_DEV

        def rows_r(c):
            return pl.ds(c * mc, hf)

        def rows_l(c):
            return pl.ds(c * mc + hf, hf)

        def rdma(src, dst, ssem, rsem, dev):
            return pltpu.make_async_remote_copy(
                src_ref=src, dst_ref=dst, send_sem=ssem, recv_sem=rsem,
                device_id=(dev,), device_id_type=pl.DeviceIdType.MESH,
            )

        barrier_sem = pltpu.get_barrier_semaphore()
        for nbr in (left, right):
            pl.semaphore_signal(
                barrier_sem, inc=1,
                device_id=(nbr,), device_id_type=pl.DeviceIdType.MESH,
            )
        pl.semaphore_wait(barrier_sem, 2)

        def gemm_chunk(c):
            acc[pl.ds(c * mc, mc), :] = jnp.dot(
                x_ref[pl.ds(c * mc, mc), :], w_ref[...],
                preferred_element_type=jnp.float32,
            ).astype(jnp.bfloat16)

        gemm_chunk(me)

        def rs_start_r(h):
            slot = h % 2
            rr = rdma(acc.at[rows_r((me - h) % N_DEV), :], comm_r.at[slot],
                      send_r.at[slot], recv_r.at[slot], right)
            rr.start()
            return rr

        def rs_start_l(h):
            slot = h % 2
            rl = rdma(acc.at[rows_l((me + h) % N_DEV), :], comm_l.at[slot],
                      send_l.at[slot], recv_l.at[slot], left)
            rl.start()
            return rl

        rr = rs_start_r(0)
        rl = rs_start_l(0)
        for k in range(1, N_DEV):
            gemm_chunk((me + k) % N_DEV)
        for h in range(N_DEV - 1):
            slot = h % 2
            rr.wait()
            rc_r = (me - h - 1) % N_DEV
            acc[rows_r(rc_r), :] = acc[rows_r(rc_r), :] + comm_r[slot]
            if h < N_DEV - 2:
                rr = rs_start_r(h + 1)
            rl.wait()
            rc_l = (me + h + 1) % N_DEV
            acc[rows_l(rc_l), :] = acc[rows_l(rc_l), :] + comm_l[slot]
            if h < N_DEV - 2:
                rl = rs_start_l(h + 1)

        scale = sx_ref[0] * sw_ref[0]
        own_r = (me + 1) % N_DEV
        own_l = (me - 1) % N_DEV
        y_r = acc[rows_r(own_r), :].astype(jnp.float32) * scale
        z_r = y_r * jax.nn.sigmoid(y_r)
        acc[rows_r(own_r), :] = z_r.astype(jnp.bfloat16)
        stage_r[0] = z_r
        y_l = acc[rows_l(own_l), :].astype(jnp.float32) * scale
        z_l = y_l * jax.nn.sigmoid(y_l)
        acc[rows_l(own_l), :] = z_l.astype(jnp.bfloat16)
        stage_l[0] = z_l

        def ag_slot(s):
            return (N_DEV - 1 + s) % 2

        def ag_start(s):
            slot = ag_slot(s)
            if s == 0:
                src_r, src_l = acc.at[rows_r(own_r), :], acc.at[rows_l(own_l), :]
            else:
                src_r, src_l = comm_r.at[ag_slot(s - 1)], comm_l.at[ag_slot(s - 1)]
            rr = rdma(src_r, comm_r.at[slot],
                      send_r.at[slot], recv_r.at[slot], right)
            rl = rdma(src_l, comm_l.at[slot],
                      send_l.at[slot], recv_l.at[slot], left)
            rr.start()
            rl.start()
            return rr, rl

        hop = ag_start(0)
        pltpu.make_async_copy(
            stage_r.at[0], out_hbm.at[rows_r(own_r), :], copy_r.at[0],
        ).start()
        pltpu.make_async_copy(
            stage_l.at[0], out_hbm.at[rows_l(own_l), :], copy_l.at[0],
        ).start()

        for s in range(N_DEV - 1):
            slot = ag_slot(s)
            cslot = (s + 1) % 2
            rr, rl = hop
            rr.wait()
            rl.wait()
            if s < N_DEV - 2:
                hop = ag_start(s + 1)
            rc_r = (me - s) % N_DEV
            rc_l = (me + s) % N_DEV
            for rows, stage, csem, cm in (
                (rows_r(rc_r), stage_r, copy_r, comm_r),
                (rows_l(rc_l), stage_l, copy_l, comm_l),
            ):
                cp = pltpu.make_async_copy(
                    stage.at[cslot], out_hbm.at[rows, :], csem.at[cslot],
                )
                if s >= 1:
                    cp.wait()
                stage[cslot] = cm[slot].astype(jnp.float32)
                cp.start()

        for cslot in (0, 1):
            for stage, csem in ((stage_r, copy_r), (stage_l, copy_l)):
                pltpu.make_async_copy(
                    stage.at[cslot], out_hbm.at[pl.ds(0, hf), :],
                    csem.at[cslot],
                ).wait()

    return pl.pallas_call(
        body,
        out_shape=jax.ShapeDtypeStruct((m, n), jnp.float32),
        in_specs=[
            pl.BlockSpec(memory_space=pltpu.VMEM),
            pl.BlockSpec(memory_space=pltpu.VMEM),
            pl.BlockSpec(memory_space=pltpu.SMEM),
            pl.BlockSpec(memory_space=pltpu.SMEM),
        ],
        out_specs=pl.BlockSpec(memory_space=pl.ANY),
        scratch_shapes=[
            pltpu.VMEM((m, n), jnp.bfloat16),
            pltpu.VMEM((2, hf, n), jnp.bfloat16),
            pltpu.VMEM((2, hf, n), jnp.bfloat16),
            pltpu.VMEM((2, hf, n), jnp.float32),
            pltpu.VMEM((2, hf, n), jnp.float32),
            pltpu.SemaphoreType.DMA((2,)),
            pltpu.SemaphoreType.DMA((2,)),
            pltpu.SemaphoreType.DMA((2,)),
            pltpu.SemaphoreType.DMA((2,)),
            pltpu.SemaphoreType.DMA((2,)),
            pltpu.SemaphoreType.DMA((2,)),
        ],
        compiler_params=pltpu.CompilerParams(
            collective_id=0,
            vmem_limit_bytes=62 * 1024 * 1024,
        ),
    )(xb, wb, scale_x, scale_w)


# device time: 200785 ns/iter; 1.7560x vs baseline; 1.0047x over previous
import jax
import jax.numpy as jnp
from jax import lax
from jax.experimental import pallas as pl
from jax.experimental.pallas import tpu as pltpu

N_DEV = 4


def kernel(x, w_mat, scale_x, scale_w):
    m, _ = x.shape
    _, n = w_mat.shape
    mc = m // N_DEV
    hf = mc // 2

    xb = x.astype(jnp.bfloat16)
    wb = w_mat.astype(jnp.bfloat16)

    def body(x_ref, w_ref, sx_ref, sw_ref, out, comm_r, comm_l,
             send_r, recv_r, send_l, recv_l):
        me = lax.axis_index("i")
        left = (me + N_DEV - 1) % N_DEV
        right = (me + 1) % N_DEV

        def rows_r(c):
            return pl.ds(c * mc, hf)

        def rows_l(c):
            return pl.ds(c * mc + hf, hf)

        def rdma(src, dst, ssem, rsem, dev):
            return pltpu.make_async_remote_copy(
                src_ref=src, dst_ref=dst, send_sem=ssem, recv_sem=rsem,
                device_id=(dev,), device_id_type=pl.DeviceIdType.MESH,
            )

        barrier_sem = pltpu.get_barrier_semaphore()
        for nbr in (left, right):
            pl.semaphore_signal(
                barrier_sem, inc=1,
                device_id=(nbr,), device_id_type=pl.DeviceIdType.MESH,
            )
        pl.semaphore_wait(barrier_sem, 2)

        def gemm_chunk(c):
            out[pl.ds(c * mc, mc), :] = jnp.dot(
                x_ref[pl.ds(c * mc, mc), :], w_ref[...],
                preferred_element_type=jnp.float32,
            ).astype(jnp.bfloat16)

        gemm_chunk(me)

        def rs_start_r(h):
            slot = h % 2
            rr = rdma(out.at[rows_r((me - h) % N_DEV), :], comm_r.at[slot],
                      send_r.at[slot], recv_r.at[slot], right)
            rr.start()
            return rr

        def rs_start_l(h):
            slot = h % 2
            rl = rdma(out.at[rows_l((me + h) % N_DEV), :], comm_l.at[slot],
                      send_l.at[slot], recv_l.at[slot], left)
            rl.start()
            return rl

        rr = rs_start_r(0)
        rl = rs_start_l(0)
        for k in range(1, N_DEV):
            gemm_chunk((me + k) % N_DEV)
        for h in range(N_DEV - 1):
            slot = h % 2
            rr.wait()
            rc_r = (me - h - 1) % N_DEV
            out[rows_r(rc_r), :] = out[rows_r(rc_r), :] + comm_r[slot]
            if h < N_DEV - 2:
                rr = rs_start_r(h + 1)
            rl.wait()
            rc_l = (me + h + 1) % N_DEV
            out[rows_l(rc_l), :] = out[rows_l(rc_l), :] + comm_l[slot]
            if h < N_DEV - 2:
                rl = rs_start_l(h + 1)

        scale = sx_ref[0] * sw_ref[0]
        own_r = (me + 1) % N_DEV
        own_l = (me - 1) % N_DEV
        y_r = out[rows_r(own_r), :].astype(jnp.float32) * scale
        out[rows_r(own_r), :] = (y_r * jax.nn.sigmoid(y_r)).astype(jnp.bfloat16)
        y_l = out[rows_l(own_l), :].astype(jnp.float32) * scale
        out[rows_l(own_l), :] = (y_l * jax.nn.sigmoid(y_l)).astype(jnp.bfloat16)

        def ag_slot(s):
            return (N_DEV - 1 + s) % 2

        def ag_start(s):
            slot = ag_slot(s)
            if s == 0:
                src_r, src_l = out.at[rows_r(own_r), :], out.at[rows_l(own_l), :]
            else:
                src_r, src_l = comm_r.at[ag_slot(s - 1)], comm_l.at[ag_slot(s - 1)]
            rr = rdma(src_r, comm_r.at[slot],
                      send_r.at[slot], recv_r.at[slot], right)
            rl = rdma(src_l, comm_l.at[slot],
                      send_l.at[slot], recv_l.at[slot], left)
            rr.start()
            rl.start()
            return rr, rl

        hop = ag_start(0)
        for s in range(N_DEV - 1):
            slot = ag_slot(s)
            rr, rl = hop
            rr.wait()
            rl.wait()
            if s < N_DEV - 2:
                hop = ag_start(s + 1)
            out[rows_r((me - s) % N_DEV), :] = comm_r[slot]
            out[rows_l((me + s) % N_DEV), :] = comm_l[slot]

    out_bf16 = pl.pallas_call(
        body,
        out_shape=jax.ShapeDtypeStruct((m, n), jnp.bfloat16),
        in_specs=[
            pl.BlockSpec(memory_space=pltpu.VMEM),
            pl.BlockSpec(memory_space=pltpu.VMEM),
            pl.BlockSpec(memory_space=pltpu.SMEM),
            pl.BlockSpec(memory_space=pltpu.SMEM),
        ],
        out_specs=pl.BlockSpec(memory_space=pltpu.VMEM),
        scratch_shapes=[
            pltpu.VMEM((2, hf, n), jnp.bfloat16),
            pltpu.VMEM((2, hf, n), jnp.bfloat16),
            pltpu.SemaphoreType.DMA((2,)),
            pltpu.SemaphoreType.DMA((2,)),
            pltpu.SemaphoreType.DMA((2,)),
            pltpu.SemaphoreType.DMA((2,)),
        ],
        compiler_params=pltpu.CompilerParams(
            collective_id=0,
            vmem_limit_bytes=60 * 1024 * 1024,
        ),
    )(xb, wb, scale_x, scale_w)
    return out_bf16.astype(jnp.float32)


# device time: 169738 ns/iter; 2.0772x vs baseline; 1.1829x over previous
import jax
import jax.numpy as jnp
from jax import lax
from jax.experimental import pallas as pl
from jax.experimental.pallas import tpu as pltpu

N_DEV = 4
S = 8


def kernel(x, w_mat, scale_x, scale_w):
    m, _ = x.shape
    _, n = w_mat.shape
    mc = m // N_DEV
    hf = mc // 2
    sub = hf // S

    k = x.shape[1]

    def body(x_ref, w_ref, sx_ref, sw_ref, out, xq, wq, comm_r, comm_l,
             send_r, recv_r, send_l, recv_l):
        me = lax.axis_index("i")
        left = (me + N_DEV - 1) % N_DEV
        right = (me + 1) % N_DEV

        def rows_r(c, i):
            return pl.ds(c * mc + i * sub, sub)

        def rows_l(c, i):
            return pl.ds(c * mc + hf + i * sub, sub)

        def comm_at(comm, slot, i):
            return comm.at[slot, pl.ds(i * sub, sub), :]

        def rdma(src, dst, ssem, rsem, dev):
            return pltpu.make_async_remote_copy(
                src_ref=src, dst_ref=dst, send_sem=ssem, recv_sem=rsem,
                device_id=(dev,), device_id_type=pl.DeviceIdType.MESH,
            )

        def gemm_chunk(c):
            xq[pl.ds(c * mc, mc), :] = x_ref[pl.ds(c * mc, mc), :].astype(
                jnp.float8_e5m2)
            for half in range(2):
                r0 = c * mc + half * hf
                out[pl.ds(r0, hf), :] = jnp.dot(
                    xq[pl.ds(r0, hf), :], wq[...],
                    preferred_element_type=jnp.float32,
                ).astype(jnp.bfloat16)

        scale = sx_ref[0] * sw_ref[0]
        own_r = (me + 1) % N_DEV
        own_l = (me - 1) % N_DEV

        def epilogue(rows):
            y = out[rows, :].astype(jnp.float32) * scale
            out[rows, :] = (y * jax.nn.sigmoid(y)).astype(jnp.bfloat16)

        def rs_start_r(h, i):
            slot = h % 2
            r = rdma(out.at[rows_r((me - h) % N_DEV, i), :],
                     comm_at(comm_r, slot, i),
                     send_r.at[slot, i], recv_r.at[slot, i], right)
            r.start()
            return r

        def rs_start_l(h, i):
            slot = h % 2
            r = rdma(out.at[rows_l((me + h) % N_DEV, i), :],
                     comm_at(comm_l, slot, i),
                     send_l.at[slot, i], recv_l.at[slot, i], left)
            r.start()
            return r

        def ag_slot(s):
            return (N_DEV - 1 + s) % 2

        def ag_start_r(s, i):
            slot = ag_slot(s)
            src = (out.at[rows_r(own_r, i), :] if s == 0
                   else comm_at(comm_r, ag_slot(s - 1), i))
            r = rdma(src, comm_at(comm_r, slot, i),
                     send_r.at[slot, i], recv_r.at[slot, i], right)
            r.start()
            return r

        def ag_start_l(s, i):
            slot = ag_slot(s)
            src = (out.at[rows_l(own_l, i), :] if s == 0
                   else comm_at(comm_l, ag_slot(s - 1), i))
            r = rdma(src, comm_at(comm_l, slot, i),
                     send_l.at[slot, i], recv_l.at[slot, i], left)
            r.start()
            return r

        barrier_sem = pltpu.get_barrier_semaphore()
        for nbr in (left, right):
            pl.semaphore_signal(
                barrier_sem, inc=1,
                device_id=(nbr,), device_id_type=pl.DeviceIdType.MESH,
            )
        wq[...] = w_ref[...].astype(jnp.float8_e5m2)
        gemm_chunk(me)
        pl.semaphore_wait(barrier_sem, 2)
        rr = [rs_start_r(0, i) for i in range(S)]
        rl = [rs_start_l(0, i) for i in range(S)]
        for k in range(1, N_DEV):
            gemm_chunk((me + k) % N_DEV)

        for h in range(N_DEV - 1):
            slot = h % 2
            rc_r = (me - h - 1) % N_DEV
            rc_l = (me + h + 1) % N_DEV
            for i in range(S):
                rr[i].wait()
                out[rows_r(rc_r, i), :] = (
                    out[rows_r(rc_r, i), :] + comm_r[slot, pl.ds(i * sub, sub), :]
                )
                if h < N_DEV - 2:
                    rr[i] = rs_start_r(h + 1, i)
                else:
                    epilogue(rows_r(own_r, i))
                    rr[i] = ag_start_r(0, i)
                rl[i].wait()
                out[rows_l(rc_l, i), :] = (
                    out[rows_l(rc_l, i), :] + comm_l[slot, pl.ds(i * sub, sub), :]
                )
                if h < N_DEV - 2:
                    rl[i] = rs_start_l(h + 1, i)
                else:
                    epilogue(rows_l(own_l, i))
                    rl[i] = ag_start_l(0, i)

        for s in range(N_DEV - 1):
            slot = ag_slot(s)
            rc_r = (me - s) % N_DEV
            rc_l = (me + s) % N_DEV
            for i in range(S):
                rr[i].wait()
                if s < N_DEV - 2:
                    rr[i] = ag_start_r(s + 1, i)
                out[rows_r(rc_r, i), :] = comm_r[slot, pl.ds(i * sub, sub), :]
                rl[i].wait()
                if s < N_DEV - 2:
                    rl[i] = ag_start_l(s + 1, i)
                out[rows_l(rc_l, i), :] = comm_l[slot, pl.ds(i * sub, sub), :]

    return pl.pallas_call(
        body,
        out_shape=jax.ShapeDtypeStruct((m, n), jnp.bfloat16),
        in_specs=[
            pl.BlockSpec(memory_space=pltpu.VMEM),
            pl.BlockSpec(memory_space=pltpu.VMEM),
            pl.BlockSpec(memory_space=pltpu.SMEM),
            pl.BlockSpec(memory_space=pltpu.SMEM),
        ],
        out_specs=pl.BlockSpec(memory_space=pltpu.VMEM),
        scratch_shapes=[
            pltpu.VMEM((m, k), jnp.float8_e5m2),
            pltpu.VMEM((k, n), jnp.float8_e5m2),
            pltpu.VMEM((2, hf, n), jnp.bfloat16),
            pltpu.VMEM((2, hf, n), jnp.bfloat16),
            pltpu.SemaphoreType.DMA((2, S)),
            pltpu.SemaphoreType.DMA((2, S)),
            pltpu.SemaphoreType.DMA((2, S)),
            pltpu.SemaphoreType.DMA((2, S)),
        ],
        compiler_params=pltpu.CompilerParams(
            collective_id=0,
            vmem_limit_bytes=62 * 1024 * 1024,
        ),
    )(x, w_mat, scale_x, scale_w)
